# baseline (speedup 1.0000x reference)
import jax
import jax.numpy as jnp
from jax.experimental import pallas as pl
from jax.experimental.pallas import tpu as pltpu

_BLK_S = 1024
_NBUF = 4


def _add_block(x_ref, t_ref, o_ref):
    o_ref[...] = x_ref[...] + t_ref[...]


def kernel(x, table):
    b, s, d = x.shape
    table_s = table[:s]

    def outer(x_hbm, t_hbm, o_hbm):
        pipeline = pltpu.emit_pipeline(
            _add_block,
            grid=(s // _BLK_S, b),
            in_specs=[
                pl.BlockSpec((1, _BLK_S, d), lambda j, i: (i, j, 0),
                             pipeline_mode=pl.Buffered(buffer_count=_NBUF)),
                pl.BlockSpec((_BLK_S, d), lambda j, i: (j, 0),
                             pipeline_mode=pl.Buffered(buffer_count=1)),
            ],
            out_specs=[
                pl.BlockSpec((1, _BLK_S, d), lambda j, i: (i, j, 0)),
            ],
        )
        pipeline(x_hbm, t_hbm, o_hbm)

    return pl.pallas_call(
        outer,
        in_specs=[
            pl.BlockSpec(memory_space=pl.ANY),
            pl.BlockSpec(memory_space=pl.ANY),
        ],
        out_specs=pl.BlockSpec(memory_space=pl.ANY),
        out_shape=jax.ShapeDtypeStruct((b, s, d), x.dtype),
    )(x, table_s)


# speedup vs baseline: 1.1129x; 1.1129x over previous
import jax
import jax.numpy as jnp
from jax.experimental import pallas as pl
from jax.experimental.pallas import tpu as pltpu

_BLK_S = 1024
_NBUF = 3


def _add_block(x_ref, t_ref, o_ref):
    o_ref[...] = x_ref[...] + t_ref[...]


def kernel(x, table):
    b, s, d = x.shape
    table_s = table[:s]

    def outer(x_hbm, t_hbm, o_hbm):
        pipeline = pltpu.emit_pipeline(
            _add_block,
            grid=(s // _BLK_S, b),
            in_specs=[
                pl.BlockSpec((1, _BLK_S, d), lambda j, i: (i, j, 0),
                             pipeline_mode=pl.Buffered(buffer_count=_NBUF)),
                pl.BlockSpec((_BLK_S, d), lambda j, i: (j, 0)),
            ],
            out_specs=[
                pl.BlockSpec((1, _BLK_S, d), lambda j, i: (i, j, 0)),
            ],
        )
        pipeline(x_hbm, t_hbm, o_hbm)

    return pl.pallas_call(
        outer,
        in_specs=[
            pl.BlockSpec(memory_space=pl.ANY),
            pl.BlockSpec(memory_space=pl.ANY),
        ],
        out_specs=pl.BlockSpec(memory_space=pl.ANY),
        out_shape=jax.ShapeDtypeStruct((b, s, d), x.dtype),
    )(x, table_s)
